# baseline (speedup 1.0000x reference)
import math

import jax
import jax.numpy as jnp
from jax import lax
from jax.experimental import pallas as pl
from jax.experimental.pallas import tpu as pltpu
from jax.experimental.pallas import tpu_sc as plsc

VOCAB = 1000000
D_MODEL = 64
B = 4096
T = 200
N_ROWS = B * T
SCALE = math.sqrt(D_MODEL)

NC = 2
NS = 16
NW = NC * NS
ROWS_PER_W = N_ROWS // NW
CHUNK = 128
N_CHUNKS = ROWS_PER_W // CHUNK
K = 4


def _sc_body(idx_hbm, table_hbm, out_hbm, idx_v, *bufs):
  gbufs = bufs[0:K]
  sbufs = bufs[K:2 * K]
  gsems = bufs[2 * K:3 * K]
  ssems = bufs[3 * K:4 * K]

  wid = lax.axis_index("s") * NC + lax.axis_index("c")
  base = wid * ROWS_PER_W

  pltpu.sync_copy(idx_hbm.at[wid], idx_v)

  def start_gather(b, g):
    pltpu.async_copy(table_hbm.at[idx_v.at[g]], gbufs[b], gsems[b])

  for b in range(K):
    start_gather(b, b)

  def group_body(go, carry):
    for b in range(K):
      g = go * K + b
      pltpu.make_async_copy(table_hbm.at[idx_v.at[g]], gbufs[b],
                            gsems[b]).wait()

      @pl.when(go > 0)
      def _wait_prev_store():
        pltpu.make_async_copy(sbufs[b], out_hbm.at[pl.ds(base, CHUNK)],
                              ssems[b]).wait()

      @plsc.parallel_loop(0, CHUNK, step=1, unroll=8)
      def _scale(r):
        for j in range(D_MODEL // 16):
          sl = pl.ds(j * 16, 16)
          sbufs[b][r, sl] = gbufs[b][r, sl] * SCALE

      pltpu.async_copy(sbufs[b], out_hbm.at[pl.ds(base + g * CHUNK, CHUNK)],
                       ssems[b])

      @pl.when(g + K < N_CHUNKS)
      def _next_gather():
        start_gather(b, g + K)
    return carry

  lax.fori_loop(0, N_CHUNKS // K, group_body, 0)

  for b in range(K):
    pltpu.make_async_copy(sbufs[b], out_hbm.at[pl.ds(base, CHUNK)],
                          ssems[b]).wait()


def kernel(indices, table):
  idx3 = indices.reshape(NW, N_CHUNKS, CHUNK)
  mesh = plsc.VectorSubcoreMesh(
      core_axis_name="c", subcore_axis_name="s", num_cores=NC,
      num_subcores=NS)
  scratch = [pltpu.VMEM((N_CHUNKS, CHUNK), jnp.int32)]
  scratch += [pltpu.VMEM((CHUNK, D_MODEL), jnp.float32) for _ in range(2 * K)]
  scratch += [pltpu.SemaphoreType.DMA for _ in range(2 * K)]
  out = pl.kernel(
      _sc_body,
      out_type=jax.ShapeDtypeStruct((N_ROWS, D_MODEL), jnp.float32),
      mesh=mesh,
      scratch_types=scratch,
      compiler_params=pltpu.CompilerParams(use_tc_tiling_on_sc=False),
  )(idx3, table)
  return out.reshape(B, T, D_MODEL)


# speedup vs baseline: 1.0547x; 1.0547x over previous
import math

import jax
import jax.numpy as jnp
from jax import lax
from jax.experimental import pallas as pl
from jax.experimental.pallas import tpu as pltpu
from jax.experimental.pallas import tpu_sc as plsc

VOCAB = 1000000
D_MODEL = 64
B = 4096
T = 200
N_ROWS = B * T
SCALE = math.sqrt(D_MODEL)

NC = 2
NS = 16
NW = NC * NS
ROWS_PER_W = N_ROWS // NW
CHUNK = 128
N_CHUNKS = ROWS_PER_W // CHUNK
K = 4


def _sc_body(idx_hbm, table_hbm, out_hbm, idx_v, *bufs):
  gbufs = bufs[0:K]
  sbufs = bufs[K:2 * K]
  gsems = bufs[2 * K:3 * K]
  ssems = bufs[3 * K:4 * K]

  wid = lax.axis_index("s") * NC + lax.axis_index("c")
  base = wid * ROWS_PER_W

  pltpu.sync_copy(idx_hbm.at[wid], idx_v)

  def start_gather(b, g):
    pltpu.async_copy(table_hbm.at[idx_v.at[g]], gbufs[b], gsems[b])

  for b in range(K):
    start_gather(b, b)

  def group_body(go, carry):
    for b in range(K):
      g = go * K + b
      pltpu.make_async_copy(table_hbm.at[idx_v.at[g]], gbufs[b],
                            gsems[b]).wait()

      @pl.when(go > 0)
      def _wait_prev_store():
        pltpu.make_async_copy(sbufs[b], out_hbm.at[pl.ds(base, CHUNK)],
                              ssems[b]).wait()

      @plsc.parallel_loop(0, CHUNK, step=1, unroll=8)
      def _scale(r):
        for j in range(D_MODEL // 16):
          sl = pl.ds(j * 16, 16)
          sbufs[b][r, sl] = gbufs[b][r, sl] * SCALE

      pltpu.async_copy(sbufs[b], out_hbm.at[pl.ds(base + g * CHUNK, CHUNK)],
                       ssems[b])

      @pl.when(g + K < N_CHUNKS)
      def _next_gather():
        start_gather(b, g + K)
    return carry

  lax.fori_loop(0, N_CHUNKS // K, group_body, 0)

  for b in range(K):
    pltpu.make_async_copy(sbufs[b], out_hbm.at[pl.ds(base, CHUNK)],
                          ssems[b]).wait()


def kernel(indices, table):
  table2 = jnp.pad(table, ((0, 0), (0, 64))).reshape(2 * VOCAB, D_MODEL)
  idx3 = (indices * 2).reshape(NW, N_CHUNKS, CHUNK)
  mesh = plsc.VectorSubcoreMesh(
      core_axis_name="c", subcore_axis_name="s", num_cores=NC,
      num_subcores=NS)
  scratch = [pltpu.VMEM((N_CHUNKS, CHUNK), jnp.int32)]
  scratch += [pltpu.VMEM((CHUNK, D_MODEL), jnp.float32) for _ in range(2 * K)]
  scratch += [pltpu.SemaphoreType.DMA for _ in range(2 * K)]
  out = pl.kernel(
      _sc_body,
      out_type=jax.ShapeDtypeStruct((N_ROWS, D_MODEL), jnp.float32),
      mesh=mesh,
      scratch_types=scratch,
      compiler_params=pltpu.CompilerParams(use_tc_tiling_on_sc=False),
  )(idx3, table2)
  return out.reshape(B, T, D_MODEL)


# speedup vs baseline: 1.0863x; 1.0300x over previous
import math

import jax
import jax.numpy as jnp
from jax import lax
from jax.experimental import pallas as pl
from jax.experimental.pallas import tpu as pltpu
from jax.experimental.pallas import tpu_sc as plsc

VOCAB = 1000000
D_MODEL = 64
B = 4096
T = 200
N_ROWS = B * T
SCALE = math.sqrt(D_MODEL)
DPAD = 128

NC = 2
NS = 16
NW = NC * NS
ROWS_PER_W = N_ROWS // NW
CHUNK = 128
N_CHUNKS = ROWS_PER_W // CHUNK


def _sc_body(idx_hbm, table_hbm, out_hbm, idx_v, gbuf, sbuf):
  wid = lax.axis_index("s") * NC + lax.axis_index("c")
  base = wid * ROWS_PER_W

  pltpu.sync_copy(idx_hbm.at[wid], idx_v)

  zeros = jnp.zeros((16,), jnp.float32)

  @plsc.parallel_loop(0, CHUNK, step=1, unroll=8)
  def _zero(r):
    for j in range(D_MODEL // 16, DPAD // 16):
      sbuf[r, pl.ds(j * 16, 16)] = zeros

  def chunk_body(g, carry):
    pltpu.sync_copy(table_hbm.at[idx_v.at[g]], gbuf)

    @plsc.parallel_loop(0, CHUNK, step=1, unroll=8)
    def _scale(r):
      for j in range(D_MODEL // 16):
        sl = pl.ds(j * 16, 16)
        sbuf[r, sl] = gbuf[r, sl] * SCALE

    pltpu.sync_copy(sbuf, out_hbm.at[pl.ds(base + g * CHUNK, CHUNK)])
    return carry

  lax.fori_loop(0, N_CHUNKS, chunk_body, 0)


def kernel(indices, table):
  table2 = jnp.pad(table, ((0, 0), (0, 64))).reshape(2 * VOCAB, D_MODEL)
  idx3 = (indices * 2).reshape(NW, N_CHUNKS, CHUNK)
  mesh = plsc.VectorSubcoreMesh(
      core_axis_name="c", subcore_axis_name="s", num_cores=NC,
      num_subcores=NS)
  scratch = [
      pltpu.VMEM((N_CHUNKS, CHUNK), jnp.int32),
      pltpu.VMEM((CHUNK, D_MODEL), jnp.float32),
      pltpu.VMEM((CHUNK, DPAD), jnp.float32),
  ]
  out = pl.kernel(
      _sc_body,
      out_type=jax.ShapeDtypeStruct((N_ROWS, DPAD), jnp.float32),
      mesh=mesh,
      scratch_types=scratch,
      compiler_params=pltpu.CompilerParams(use_tc_tiling_on_sc=False),
  )(idx3, table2)
  return out[:, :D_MODEL].reshape(B, T, D_MODEL)
